# Initial kernel scaffold; baseline (speedup 1.0000x reference)
import jax
import jax.numpy as jnp
from jax.experimental import pallas as pl


def kernel(pred_deltas, pred_labels, prior_boxes):
    raise NotImplementedError("write your pallas kernel here")


# speedup vs baseline: 3.0954x; 3.0954x over previous
import jax
import jax.numpy as jnp
from jax.experimental import pallas as pl
from jax.experimental.pallas import tpu as pltpu

_K = 200
_IOU_THR = 0.5
_SCORE_THR = 0.5
_VAR = (0.1, 0.1, 0.2, 0.2)
_NEG = float("-inf")


def _nms_kernel(labels_ref, deltas_ref, priors_ref,
                y1o, x1o, y2o, x2o, sco, clso,
                sc_ref, ky_ref, sy1, sx1, sy2, sx2):
    bb, c, n = labels_ref.shape

    py1 = priors_ref[:, 0:1, :]
    px1 = priors_ref[:, 1:2, :]
    py2 = priors_ref[:, 2:3, :]
    px2 = priors_ref[:, 3:4, :]
    anc_h = py2 - py1
    anc_w = px2 - px1
    ctr_y = py1 + 0.5 * anc_h
    ctr_x = px1 + 0.5 * anc_w
    d0 = deltas_ref[:, 0:1, :] * _VAR[0]
    d1 = deltas_ref[:, 1:2, :] * _VAR[1]
    d2 = deltas_ref[:, 2:3, :] * _VAR[2]
    d3 = deltas_ref[:, 3:4, :] * _VAR[3]
    bh = jnp.exp(d2) * anc_h
    bw = jnp.exp(d3) * anc_w
    bcy = d0 * anc_h + ctr_y
    bcx = d1 * anc_w + ctr_x
    y1 = bcy - 0.5 * bh
    x1 = bcx - 0.5 * bw
    y2 = y1 + bh
    x2 = x1 + bw
    areas = (y2 - y1) * (x2 - x1)

    labels = labels_ref[...]
    colmax = jnp.max(labels, axis=1, keepdims=True)
    bg = labels[:, 0:1, :] >= colmax
    s = jnp.where(bg, 0.0, labels)
    sc_ref[...] = jnp.where(s > _SCORE_THR, s, _NEG)

    lane = jax.lax.broadcasted_iota(jnp.int32, (bb, c, n), 2)
    slot = jax.lax.broadcasted_iota(jnp.int32, (bb, c, _K), 2)

    ky_ref[...] = jnp.full((bb, c, _K), -1.0, jnp.float32)
    sy1[...] = jnp.zeros((bb, c, _K), jnp.float32)
    sx1[...] = jnp.zeros((bb, c, _K), jnp.float32)
    sy2[...] = jnp.zeros((bb, c, _K), jnp.float32)
    sx2[...] = jnp.zeros((bb, c, _K), jnp.float32)

    def nms_body(t, carry):
        sc = sc_ref[...]
        m = jnp.max(sc, axis=2, keepdims=True)
        valid = m > _NEG
        idx = jnp.min(jnp.where(sc == m, lane, n), axis=2, keepdims=True)
        onehot = lane == idx
        by1 = jnp.sum(jnp.where(onehot, y1, 0.0), axis=2, keepdims=True)
        bx1 = jnp.sum(jnp.where(onehot, x1, 0.0), axis=2, keepdims=True)
        by2 = jnp.sum(jnp.where(onehot, y2, 0.0), axis=2, keepdims=True)
        bx2 = jnp.sum(jnp.where(onehot, x2, 0.0), axis=2, keepdims=True)
        a = (by2 - by1) * (bx2 - bx1)
        yy1 = jnp.maximum(by1, y1)
        xx1 = jnp.maximum(bx1, x1)
        yy2 = jnp.minimum(by2, y2)
        xx2 = jnp.minimum(bx2, x2)
        inter = jnp.maximum(yy2 - yy1, 0.0) * jnp.maximum(xx2 - xx1, 0.0)
        iou = inter / (a + areas - inter + 1e-8)
        supp = (iou > _IOU_THR) | onehot
        sc_ref[...] = jnp.where(valid & supp, _NEG, sc)
        wmask = (slot == t) & valid
        ky_ref[...] = jnp.where(wmask, m, ky_ref[...])
        sy1[...] = jnp.where(wmask, by1, sy1[...])
        sx1[...] = jnp.where(wmask, bx1, sx1[...])
        sy2[...] = jnp.where(wmask, by2, sy2[...])
        sx2[...] = jnp.where(wmask, bx2, sx2[...])
        return carry

    jax.lax.fori_loop(0, _K, nms_body, 0)

    fi = (jax.lax.broadcasted_iota(jnp.int32, (bb, c, _K), 1) * _K
          + jax.lax.broadcasted_iota(jnp.int32, (bb, c, _K), 2))
    oslot = jax.lax.broadcasted_iota(jnp.int32, (1, bb, _K), 2)

    def merge_body(t, carry):
        ky = ky_ref[...]
        m = jnp.max(ky, axis=(1, 2), keepdims=True)
        pos = jnp.min(jnp.where(ky == m, fi, c * _K), axis=(1, 2), keepdims=True)
        oh = fi == pos
        valid = m > 0.0
        omask = oslot == t
        sco[...] = jnp.where(omask, jnp.where(valid, m, 0.0)[:, :, 0][None], sco[...])
        cls = (pos // _K).astype(jnp.float32)
        clso[...] = jnp.where(omask, jnp.where(valid, cls, 0.0)[:, :, 0][None], clso[...])
        by1 = jnp.sum(jnp.where(oh, sy1[...], 0.0), axis=(1, 2), keepdims=True)
        bx1 = jnp.sum(jnp.where(oh, sx1[...], 0.0), axis=(1, 2), keepdims=True)
        by2 = jnp.sum(jnp.where(oh, sy2[...], 0.0), axis=(1, 2), keepdims=True)
        bx2 = jnp.sum(jnp.where(oh, sx2[...], 0.0), axis=(1, 2), keepdims=True)
        y1o[...] = jnp.where(omask, by1[:, :, 0][None], y1o[...])
        x1o[...] = jnp.where(omask, bx1[:, :, 0][None], x1o[...])
        y2o[...] = jnp.where(omask, by2[:, :, 0][None], y2o[...])
        x2o[...] = jnp.where(omask, bx2[:, :, 0][None], x2o[...])
        ky_ref[...] = jnp.where(oh, -2.0, ky)
        return carry

    jax.lax.fori_loop(0, _K, merge_body, 0)


def kernel(pred_deltas, pred_labels, prior_boxes):
    b, n, c = pred_labels.shape
    npad = -n % 128
    nn = n + npad
    labels_t = jnp.pad(pred_labels.transpose(0, 2, 1), ((0, 0), (0, 0), (0, npad)))
    deltas_t = jnp.pad(pred_deltas.transpose(0, 2, 1), ((0, 0), (0, 0), (0, npad)))
    priors_t = jnp.pad(prior_boxes.T, ((0, 0), (0, npad)))[None]

    ncores = 2 if b % 2 == 0 else 1
    bb = b // ncores
    f32 = jnp.float32
    outs = pl.pallas_call(
        _nms_kernel,
        grid=(ncores,),
        in_specs=[
            pl.BlockSpec((bb, c, nn), lambda i: (i, 0, 0)),
            pl.BlockSpec((bb, 4, nn), lambda i: (i, 0, 0)),
            pl.BlockSpec((1, 4, nn), lambda i: (0, 0, 0)),
        ],
        out_specs=[pl.BlockSpec((1, bb, _K), lambda i: (i, 0, 0))] * 6,
        out_shape=[jax.ShapeDtypeStruct((ncores, bb, _K), f32)] * 6,
        scratch_shapes=[
            pltpu.VMEM((bb, c, nn), f32),
            pltpu.VMEM((bb, c, _K), f32),
            pltpu.VMEM((bb, c, _K), f32),
            pltpu.VMEM((bb, c, _K), f32),
            pltpu.VMEM((bb, c, _K), f32),
            pltpu.VMEM((bb, c, _K), f32),
        ],
        compiler_params=pltpu.CompilerParams(dimension_semantics=("parallel",)),
    )(labels_t, deltas_t, priors_t)
    y1, x1, y2, x2, scores, cls = [o.reshape(b, _K) for o in outs]
    boxes = jnp.stack([y1, x1, y2, x2], axis=-1)
    return boxes, scores, cls


# speedup vs baseline: 16.5356x; 5.3420x over previous
import jax
import jax.numpy as jnp
from jax.experimental import pallas as pl
from jax.experimental.pallas import tpu as pltpu

_K = 200
_IOU_THR = 0.5
_SCORE_THR = 0.5
_VAR = (0.1, 0.1, 0.2, 0.2)
_NEG = float("-inf")


def _nms_kernel(labels_ref, deltas_ref, priors_ref,
                y1o, x1o, y2o, x2o, sco, clso,
                sc_ref, ky_ref, sy1, sx1, sy2, sx2):
    bb, c, n = labels_ref.shape

    py1 = priors_ref[:, 0:1, :]
    px1 = priors_ref[:, 1:2, :]
    py2 = priors_ref[:, 2:3, :]
    px2 = priors_ref[:, 3:4, :]
    anc_h = py2 - py1
    anc_w = px2 - px1
    ctr_y = py1 + 0.5 * anc_h
    ctr_x = px1 + 0.5 * anc_w
    d0 = deltas_ref[:, 0:1, :] * _VAR[0]
    d1 = deltas_ref[:, 1:2, :] * _VAR[1]
    d2 = deltas_ref[:, 2:3, :] * _VAR[2]
    d3 = deltas_ref[:, 3:4, :] * _VAR[3]
    bh = jnp.exp(d2) * anc_h
    bw = jnp.exp(d3) * anc_w
    bcy = d0 * anc_h + ctr_y
    bcx = d1 * anc_w + ctr_x
    y1 = bcy - 0.5 * bh
    x1 = bcx - 0.5 * bw
    y2 = y1 + bh
    x2 = x1 + bw
    areas = (y2 - y1) * (x2 - x1)

    labels = labels_ref[...]
    colmax = jnp.max(labels, axis=1, keepdims=True)
    bg = labels[:, 0:1, :] >= colmax
    s = jnp.where(bg, 0.0, labels)
    sc_ref[...] = jnp.where(s > _SCORE_THR, s, _NEG)

    lane = jax.lax.broadcasted_iota(jnp.int32, (bb, c, n), 2)
    slot = jax.lax.broadcasted_iota(jnp.int32, (bb, c, _K), 2)

    ky_ref[...] = jnp.full((bb, c, _K), -1.0, jnp.float32)
    sy1[...] = jnp.zeros((bb, c, _K), jnp.float32)
    sx1[...] = jnp.zeros((bb, c, _K), jnp.float32)
    sy2[...] = jnp.zeros((bb, c, _K), jnp.float32)
    sx2[...] = jnp.zeros((bb, c, _K), jnp.float32)

    def nms_cond(carry):
        t, done = carry
        return jnp.logical_and(t < _K, jnp.logical_not(done))

    def nms_body(carry):
        t, _ = carry
        sc = sc_ref[...]
        m = jnp.max(sc, axis=2, keepdims=True)
        valid = m > _NEG
        idx = jnp.min(jnp.where(sc == m, lane, n), axis=2, keepdims=True)
        onehot = lane == idx
        by1 = jnp.sum(jnp.where(onehot, y1, 0.0), axis=2, keepdims=True)
        bx1 = jnp.sum(jnp.where(onehot, x1, 0.0), axis=2, keepdims=True)
        by2 = jnp.sum(jnp.where(onehot, y2, 0.0), axis=2, keepdims=True)
        bx2 = jnp.sum(jnp.where(onehot, x2, 0.0), axis=2, keepdims=True)
        a = (by2 - by1) * (bx2 - bx1)
        yy1 = jnp.maximum(by1, y1)
        xx1 = jnp.maximum(bx1, x1)
        yy2 = jnp.minimum(by2, y2)
        xx2 = jnp.minimum(bx2, x2)
        inter = jnp.maximum(yy2 - yy1, 0.0) * jnp.maximum(xx2 - xx1, 0.0)
        iou = inter / (a + areas - inter + 1e-8)
        supp = (iou > _IOU_THR) | onehot
        sc_ref[...] = jnp.where(valid & supp, _NEG, sc)
        wmask = (slot == t) & valid
        ky_ref[...] = jnp.where(wmask, m, ky_ref[...])
        sy1[...] = jnp.where(wmask, by1, sy1[...])
        sx1[...] = jnp.where(wmask, bx1, sx1[...])
        sy2[...] = jnp.where(wmask, by2, sy2[...])
        sx2[...] = jnp.where(wmask, bx2, sx2[...])
        big_m = jnp.max(m, axis=1, keepdims=True)
        cnt = jnp.sum((ky_ref[...] > big_m).astype(jnp.int32),
                      axis=(1, 2), keepdims=True)
        batch_done = (cnt >= _K) | (big_m <= _NEG)
        return t + 1, jnp.all(batch_done)

    jax.lax.while_loop(nms_cond, nms_body, (0, False))

    fi = (jax.lax.broadcasted_iota(jnp.int32, (bb, c, _K), 1) * _K
          + jax.lax.broadcasted_iota(jnp.int32, (bb, c, _K), 2))
    oslot = jax.lax.broadcasted_iota(jnp.int32, (1, bb, _K), 2)

    def merge_body(t, carry):
        ky = ky_ref[...]
        m = jnp.max(ky, axis=(1, 2), keepdims=True)
        pos = jnp.min(jnp.where(ky == m, fi, c * _K), axis=(1, 2), keepdims=True)
        oh = fi == pos
        valid = m > 0.0
        omask = oslot == t
        sco[...] = jnp.where(omask, jnp.where(valid, m, 0.0)[:, :, 0][None], sco[...])
        cls = (pos // _K).astype(jnp.float32)
        clso[...] = jnp.where(omask, jnp.where(valid, cls, 0.0)[:, :, 0][None], clso[...])
        by1 = jnp.sum(jnp.where(oh, sy1[...], 0.0), axis=(1, 2), keepdims=True)
        bx1 = jnp.sum(jnp.where(oh, sx1[...], 0.0), axis=(1, 2), keepdims=True)
        by2 = jnp.sum(jnp.where(oh, sy2[...], 0.0), axis=(1, 2), keepdims=True)
        bx2 = jnp.sum(jnp.where(oh, sx2[...], 0.0), axis=(1, 2), keepdims=True)
        y1o[...] = jnp.where(omask, by1[:, :, 0][None], y1o[...])
        x1o[...] = jnp.where(omask, bx1[:, :, 0][None], x1o[...])
        y2o[...] = jnp.where(omask, by2[:, :, 0][None], y2o[...])
        x2o[...] = jnp.where(omask, bx2[:, :, 0][None], x2o[...])
        ky_ref[...] = jnp.where(oh, -2.0, ky)
        return carry

    jax.lax.fori_loop(0, _K, merge_body, 0)


def kernel(pred_deltas, pred_labels, prior_boxes):
    b, n, c = pred_labels.shape
    npad = -n % 128
    nn = n + npad
    labels_t = jnp.pad(pred_labels.transpose(0, 2, 1), ((0, 0), (0, 0), (0, npad)))
    deltas_t = jnp.pad(pred_deltas.transpose(0, 2, 1), ((0, 0), (0, 0), (0, npad)))
    priors_t = jnp.pad(prior_boxes.T, ((0, 0), (0, npad)))[None]

    ncores = 2 if b % 2 == 0 else 1
    bb = b // ncores
    f32 = jnp.float32
    outs = pl.pallas_call(
        _nms_kernel,
        grid=(ncores,),
        in_specs=[
            pl.BlockSpec((bb, c, nn), lambda i: (i, 0, 0)),
            pl.BlockSpec((bb, 4, nn), lambda i: (i, 0, 0)),
            pl.BlockSpec((1, 4, nn), lambda i: (0, 0, 0)),
        ],
        out_specs=[pl.BlockSpec((1, bb, _K), lambda i: (i, 0, 0))] * 6,
        out_shape=[jax.ShapeDtypeStruct((ncores, bb, _K), f32)] * 6,
        scratch_shapes=[
            pltpu.VMEM((bb, c, nn), f32),
            pltpu.VMEM((bb, c, _K), f32),
            pltpu.VMEM((bb, c, _K), f32),
            pltpu.VMEM((bb, c, _K), f32),
            pltpu.VMEM((bb, c, _K), f32),
            pltpu.VMEM((bb, c, _K), f32),
        ],
        compiler_params=pltpu.CompilerParams(dimension_semantics=("parallel",)),
    )(labels_t, deltas_t, priors_t)
    y1, x1, y2, x2, scores, cls = [o.reshape(b, _K) for o in outs]
    boxes = jnp.stack([y1, x1, y2, x2], axis=-1)
    return boxes, scores, cls


# speedup vs baseline: 18.5164x; 1.1198x over previous
import jax
import jax.numpy as jnp
from jax.experimental import pallas as pl
from jax.experimental.pallas import tpu as pltpu

_K = 200
_IOU_THR = 0.5
_SCORE_THR = 0.5
_VAR = (0.1, 0.1, 0.2, 0.2)
_NEG = float("-inf")


def _nms_kernel(labels_ref, deltas_ref, priors_ref,
                y1o, x1o, y2o, x2o, sco, clso,
                sc_ref, ky_ref, sy1, sx1, sy2, sx2):
    bb, c, n = labels_ref.shape

    py1 = priors_ref[:, 0:1, :]
    px1 = priors_ref[:, 1:2, :]
    py2 = priors_ref[:, 2:3, :]
    px2 = priors_ref[:, 3:4, :]
    anc_h = py2 - py1
    anc_w = px2 - px1
    ctr_y = py1 + 0.5 * anc_h
    ctr_x = px1 + 0.5 * anc_w
    d0 = deltas_ref[:, 0:1, :] * _VAR[0]
    d1 = deltas_ref[:, 1:2, :] * _VAR[1]
    d2 = deltas_ref[:, 2:3, :] * _VAR[2]
    d3 = deltas_ref[:, 3:4, :] * _VAR[3]
    bh = jnp.exp(d2) * anc_h
    bw = jnp.exp(d3) * anc_w
    bcy = d0 * anc_h + ctr_y
    bcx = d1 * anc_w + ctr_x
    y1 = bcy - 0.5 * bh
    x1 = bcx - 0.5 * bw
    y2 = y1 + bh
    x2 = x1 + bw
    areas = (y2 - y1) * (x2 - x1)

    labels = labels_ref[...]
    colmax = jnp.max(labels, axis=1, keepdims=True)
    bg = labels[:, 0:1, :] >= colmax
    s = jnp.where(bg, 0.0, labels)
    sc_ref[...] = jnp.where(s > _SCORE_THR, s, _NEG)

    lane = jax.lax.broadcasted_iota(jnp.int32, (bb, c, n), 2)
    slot = jax.lax.broadcasted_iota(jnp.int32, (bb, c, _K), 2)

    ky_ref[...] = jnp.full((bb, c, _K), -1.0, jnp.float32)
    sy1[...] = jnp.zeros((bb, c, _K), jnp.float32)
    sx1[...] = jnp.zeros((bb, c, _K), jnp.float32)
    sy2[...] = jnp.zeros((bb, c, _K), jnp.float32)
    sx2[...] = jnp.zeros((bb, c, _K), jnp.float32)

    def nms_cond(carry):
        t, done = carry
        return jnp.logical_and(t < _K, jnp.logical_not(done))

    def nms_body(carry):
        t, _ = carry
        sc = sc_ref[...]
        m = jnp.max(sc, axis=2, keepdims=True)
        valid = m > _NEG
        idx = jnp.min(jnp.where(sc == m, lane, n), axis=2, keepdims=True)
        onehot = lane == idx
        by1 = jnp.sum(jnp.where(onehot, y1, 0.0), axis=2, keepdims=True)
        bx1 = jnp.sum(jnp.where(onehot, x1, 0.0), axis=2, keepdims=True)
        by2 = jnp.sum(jnp.where(onehot, y2, 0.0), axis=2, keepdims=True)
        bx2 = jnp.sum(jnp.where(onehot, x2, 0.0), axis=2, keepdims=True)
        a = (by2 - by1) * (bx2 - bx1)
        yy1 = jnp.maximum(by1, y1)
        xx1 = jnp.maximum(bx1, x1)
        yy2 = jnp.minimum(by2, y2)
        xx2 = jnp.minimum(bx2, x2)
        inter = jnp.maximum(yy2 - yy1, 0.0) * jnp.maximum(xx2 - xx1, 0.0)
        iou = inter / (a + areas - inter + 1e-8)
        supp = (iou > _IOU_THR) | onehot
        sc_ref[...] = jnp.where(valid & supp, _NEG, sc)
        wmask = (slot == t) & valid
        ky_ref[...] = jnp.where(wmask, m, ky_ref[...])
        sy1[...] = jnp.where(wmask, by1, sy1[...])
        sx1[...] = jnp.where(wmask, bx1, sx1[...])
        sy2[...] = jnp.where(wmask, by2, sy2[...])
        sx2[...] = jnp.where(wmask, bx2, sx2[...])
        big_m = jnp.max(m, axis=1, keepdims=True)
        cnt = jnp.sum((ky_ref[...] > big_m).astype(jnp.int32),
                      axis=(1, 2), keepdims=True)
        batch_done = (cnt >= _K) | (big_m <= _NEG)
        return t + 1, jnp.all(batch_done)

    jax.lax.while_loop(nms_cond, nms_body, (0, False))

    fi = (jax.lax.broadcasted_iota(jnp.int32, (bb, c, _K), 1) * _K
          + jax.lax.broadcasted_iota(jnp.int32, (bb, c, _K), 2))
    oslot = jax.lax.broadcasted_iota(jnp.int32, (1, bb, _K), 2)

    unroll = 4
    tv1, tx1, tv2, tx2 = sy1[...], sx1[...], sy2[...], sx2[...]

    def merge_body(i, carry):
        ky = ky_ref[...]
        for u in range(unroll):
            t = i * unroll + u
            m = jnp.max(ky, axis=(1, 2), keepdims=True)
            pos = jnp.min(jnp.where(ky == m, fi, c * _K), axis=(1, 2), keepdims=True)
            oh = fi == pos
            valid = m > 0.0
            omask = oslot == t
            sco[...] = jnp.where(omask, jnp.where(valid, m, 0.0)[:, :, 0][None], sco[...])
            cls = (pos // _K).astype(jnp.float32)
            clso[...] = jnp.where(omask, jnp.where(valid, cls, 0.0)[:, :, 0][None], clso[...])
            by1 = jnp.sum(jnp.where(oh, tv1, 0.0), axis=(1, 2), keepdims=True)
            bx1 = jnp.sum(jnp.where(oh, tx1, 0.0), axis=(1, 2), keepdims=True)
            by2 = jnp.sum(jnp.where(oh, tv2, 0.0), axis=(1, 2), keepdims=True)
            bx2 = jnp.sum(jnp.where(oh, tx2, 0.0), axis=(1, 2), keepdims=True)
            y1o[...] = jnp.where(omask, by1[:, :, 0][None], y1o[...])
            x1o[...] = jnp.where(omask, bx1[:, :, 0][None], x1o[...])
            y2o[...] = jnp.where(omask, by2[:, :, 0][None], y2o[...])
            x2o[...] = jnp.where(omask, bx2[:, :, 0][None], x2o[...])
            ky = jnp.where(oh, -2.0, ky)
        ky_ref[...] = ky
        return carry

    jax.lax.fori_loop(0, _K // unroll, merge_body, 0)


def kernel(pred_deltas, pred_labels, prior_boxes):
    b, n, c = pred_labels.shape
    npad = -n % 128
    nn = n + npad
    labels_t = jnp.pad(pred_labels.transpose(0, 2, 1), ((0, 0), (0, 0), (0, npad)))
    deltas_t = jnp.pad(pred_deltas.transpose(0, 2, 1), ((0, 0), (0, 0), (0, npad)))
    priors_t = jnp.pad(prior_boxes.T, ((0, 0), (0, npad)))[None]

    ncores = 2 if b % 2 == 0 else 1
    bb = b // ncores
    f32 = jnp.float32
    outs = pl.pallas_call(
        _nms_kernel,
        grid=(ncores,),
        in_specs=[
            pl.BlockSpec((bb, c, nn), lambda i: (i, 0, 0)),
            pl.BlockSpec((bb, 4, nn), lambda i: (i, 0, 0)),
            pl.BlockSpec((1, 4, nn), lambda i: (0, 0, 0)),
        ],
        out_specs=[pl.BlockSpec((1, bb, _K), lambda i: (i, 0, 0))] * 6,
        out_shape=[jax.ShapeDtypeStruct((ncores, bb, _K), f32)] * 6,
        scratch_shapes=[
            pltpu.VMEM((bb, c, nn), f32),
            pltpu.VMEM((bb, c, _K), f32),
            pltpu.VMEM((bb, c, _K), f32),
            pltpu.VMEM((bb, c, _K), f32),
            pltpu.VMEM((bb, c, _K), f32),
            pltpu.VMEM((bb, c, _K), f32),
        ],
        compiler_params=pltpu.CompilerParams(dimension_semantics=("parallel",)),
    )(labels_t, deltas_t, priors_t)
    y1, x1, y2, x2, scores, cls = [o.reshape(b, _K) for o in outs]
    boxes = jnp.stack([y1, x1, y2, x2], axis=-1)
    return boxes, scores, cls


# speedup vs baseline: 27.3884x; 1.4791x over previous
import jax
import jax.numpy as jnp
from jax.experimental import pallas as pl
from jax.experimental.pallas import tpu as pltpu

_K = 200
_MW = 32
_IOU_THR = 0.5
_SCORE_THR = 0.5
_VAR = (0.1, 0.1, 0.2, 0.2)
_NEG = float("-inf")


def _nms_kernel(labels_ref, deltas_ref, priors_ref,
                y1o, x1o, y2o, x2o, sco, clso,
                sc_ref, ky_ref, sy1, sx1, sy2, sx2):
    bb, c, n = labels_ref.shape

    py1 = priors_ref[:, 0:1, :]
    px1 = priors_ref[:, 1:2, :]
    py2 = priors_ref[:, 2:3, :]
    px2 = priors_ref[:, 3:4, :]
    anc_h = py2 - py1
    anc_w = px2 - px1
    ctr_y = py1 + 0.5 * anc_h
    ctr_x = px1 + 0.5 * anc_w
    d0 = deltas_ref[:, 0:1, :] * _VAR[0]
    d1 = deltas_ref[:, 1:2, :] * _VAR[1]
    d2 = deltas_ref[:, 2:3, :] * _VAR[2]
    d3 = deltas_ref[:, 3:4, :] * _VAR[3]
    bh = jnp.exp(d2) * anc_h
    bw = jnp.exp(d3) * anc_w
    bcy = d0 * anc_h + ctr_y
    bcx = d1 * anc_w + ctr_x
    y1 = bcy - 0.5 * bh
    x1 = bcx - 0.5 * bw
    y2 = y1 + bh
    x2 = x1 + bw
    areas = (y2 - y1) * (x2 - x1)

    labels = labels_ref[...]
    colmax = jnp.max(labels, axis=1, keepdims=True)
    bg = labels[:, 0:1, :] >= colmax
    s = jnp.where(bg, 0.0, labels)
    sc_ref[...] = jnp.where(s > _SCORE_THR, s, _NEG)

    lane = jax.lax.broadcasted_iota(jnp.int32, (bb, c, n), 2)
    slot = jax.lax.broadcasted_iota(jnp.int32, (bb, c, _K), 2)

    ky_ref[...] = jnp.full((bb, c, _K), -1.0, jnp.float32)
    sy1[...] = jnp.zeros((bb, c, _K), jnp.float32)
    sx1[...] = jnp.zeros((bb, c, _K), jnp.float32)
    sy2[...] = jnp.zeros((bb, c, _K), jnp.float32)
    sx2[...] = jnp.zeros((bb, c, _K), jnp.float32)

    def nms_cond(carry):
        t, done = carry
        return jnp.logical_and(t < _K, jnp.logical_not(done))

    def nms_body(carry):
        t, _ = carry
        sc = sc_ref[...]
        m = jnp.max(sc, axis=2, keepdims=True)
        valid = m > _NEG
        idx = jnp.min(jnp.where(sc == m, lane, n), axis=2, keepdims=True)
        onehot = lane == idx
        by1 = jnp.sum(jnp.where(onehot, y1, 0.0), axis=2, keepdims=True)
        bx1 = jnp.sum(jnp.where(onehot, x1, 0.0), axis=2, keepdims=True)
        by2 = jnp.sum(jnp.where(onehot, y2, 0.0), axis=2, keepdims=True)
        bx2 = jnp.sum(jnp.where(onehot, x2, 0.0), axis=2, keepdims=True)
        a = (by2 - by1) * (bx2 - bx1)
        yy1 = jnp.maximum(by1, y1)
        xx1 = jnp.maximum(bx1, x1)
        yy2 = jnp.minimum(by2, y2)
        xx2 = jnp.minimum(bx2, x2)
        inter = jnp.maximum(yy2 - yy1, 0.0) * jnp.maximum(xx2 - xx1, 0.0)
        iou = inter / (a + areas - inter + 1e-8)
        supp = (iou > _IOU_THR) | onehot
        sc_ref[...] = jnp.where(valid & supp, _NEG, sc)
        wmask = (slot == t) & valid
        ky_ref[...] = jnp.where(wmask, m, ky_ref[...])
        sy1[...] = jnp.where(wmask, by1, sy1[...])
        sx1[...] = jnp.where(wmask, bx1, sx1[...])
        sy2[...] = jnp.where(wmask, by2, sy2[...])
        sx2[...] = jnp.where(wmask, bx2, sx2[...])
        big_m = jnp.max(m, axis=1, keepdims=True)
        cnt = jnp.sum((ky_ref[...] > big_m).astype(jnp.int32),
                      axis=(1, 2), keepdims=True)
        batch_done = (cnt >= _K) | (big_m <= _NEG)
        return t + 1, jnp.all(batch_done)

    t_stop, _ = jax.lax.while_loop(nms_cond, nms_body, (0, False))

    w = min(_MW, _K)
    sl = 1
    while sl < c:
        sl *= 2
    nfl = sl * w
    s_io = jax.lax.broadcasted_iota(jnp.int32, (bb, sl, w), 1)
    l_io = jax.lax.broadcasted_iota(jnp.int32, (bb, sl, w), 2)
    ffl = s_io * w + l_io
    fi3 = s_io * _K + l_io

    def padrows(x, fill):
        if sl == c:
            return x
        return jnp.concatenate(
            [x, jnp.full((bb, sl - c, w), fill, jnp.float32)], axis=1)

    key = padrows(ky_ref[:, :, :w], -1.0)
    pv1 = padrows(sy1[:, :, :w], 0.0)
    px1 = padrows(sx1[:, :, :w], 0.0)
    pv2 = padrows(sy2[:, :, :w], 0.0)
    px2 = padrows(sx2[:, :, :w], 0.0)
    fis = fi3

    k = 2
    while k <= nfl:
        j = k // 2
        while j >= 1:
            if j < w:
                ax, sh, lower = 2, j, (l_io & j) == 0
            else:
                ax, sh, lower = 1, j // w, (s_io & (j // w)) == 0
            seg0 = (ffl & k) == 0

            def ex(x, _ax=ax, _sh=sh, _lo=lower):
                return jnp.where(_lo, jnp.roll(x, -_sh, axis=_ax),
                                 jnp.roll(x, _sh, axis=_ax))

            okey, ofis = ex(key), ex(fis)
            g = (key > okey) | ((key == okey) & (fis < ofis))
            sel_mine = (seg0 == lower) == g
            key = jnp.where(sel_mine, key, okey)
            fis = jnp.where(sel_mine, fis, ofis)
            pv1 = jnp.where(sel_mine, pv1, ex(pv1))
            px1 = jnp.where(sel_mine, px1, ex(px1))
            pv2 = jnp.where(sel_mine, pv2, ex(pv2))
            px2 = jnp.where(sel_mine, px2, ex(px2))
            j //= 2
        k *= 2

    fast = t_stop <= w
    fvalid = key > 0.0
    fsc = jnp.where(fvalid, key, 0.0)
    fcls = jnp.where(fvalid, (fis // _K).astype(jnp.float32), 0.0)
    row = 0
    off = 0
    while off < _K:
        ln = min(w, _K - off)
        osl = slice(off, off + ln)
        sco[0, :, osl] = jnp.where(fast, fsc[:, row, :ln], sco[0, :, osl])
        clso[0, :, osl] = jnp.where(fast, fcls[:, row, :ln], clso[0, :, osl])
        y1o[0, :, osl] = jnp.where(fast, pv1[:, row, :ln], y1o[0, :, osl])
        x1o[0, :, osl] = jnp.where(fast, px1[:, row, :ln], x1o[0, :, osl])
        y2o[0, :, osl] = jnp.where(fast, pv2[:, row, :ln], y2o[0, :, osl])
        x2o[0, :, osl] = jnp.where(fast, px2[:, row, :ln], x2o[0, :, osl])
        row += 1
        off += ln

    fi = (jax.lax.broadcasted_iota(jnp.int32, (bb, c, _K), 1) * _K
          + jax.lax.broadcasted_iota(jnp.int32, (bb, c, _K), 2))
    oslot = jax.lax.broadcasted_iota(jnp.int32, (1, bb, _K), 2)

    unroll = 4
    tv1, tx1, tv2, tx2 = sy1[...], sx1[...], sy2[...], sx2[...]

    def merge_body(i, carry):
        ky = ky_ref[...]
        for u in range(unroll):
            t = i * unroll + u
            m = jnp.max(ky, axis=(1, 2), keepdims=True)
            pos = jnp.min(jnp.where(ky == m, fi, c * _K), axis=(1, 2), keepdims=True)
            oh = fi == pos
            valid = m > 0.0
            omask = oslot == t
            sco[...] = jnp.where(omask, jnp.where(valid, m, 0.0)[:, :, 0][None], sco[...])
            cls = (pos // _K).astype(jnp.float32)
            clso[...] = jnp.where(omask, jnp.where(valid, cls, 0.0)[:, :, 0][None], clso[...])
            by1 = jnp.sum(jnp.where(oh, tv1, 0.0), axis=(1, 2), keepdims=True)
            bx1 = jnp.sum(jnp.where(oh, tx1, 0.0), axis=(1, 2), keepdims=True)
            by2 = jnp.sum(jnp.where(oh, tv2, 0.0), axis=(1, 2), keepdims=True)
            bx2 = jnp.sum(jnp.where(oh, tx2, 0.0), axis=(1, 2), keepdims=True)
            y1o[...] = jnp.where(omask, by1[:, :, 0][None], y1o[...])
            x1o[...] = jnp.where(omask, bx1[:, :, 0][None], x1o[...])
            y2o[...] = jnp.where(omask, by2[:, :, 0][None], y2o[...])
            x2o[...] = jnp.where(omask, bx2[:, :, 0][None], x2o[...])
            ky = jnp.where(oh, -2.0, ky)
        ky_ref[...] = ky
        return carry

    jax.lax.fori_loop(0, jnp.where(fast, 0, _K // unroll), merge_body, 0)


def kernel(pred_deltas, pred_labels, prior_boxes):
    b, n, c = pred_labels.shape
    npad = -n % 128
    nn = n + npad
    labels_t = jnp.pad(pred_labels.transpose(0, 2, 1), ((0, 0), (0, 0), (0, npad)))
    deltas_t = jnp.pad(pred_deltas.transpose(0, 2, 1), ((0, 0), (0, 0), (0, npad)))
    priors_t = jnp.pad(prior_boxes.T, ((0, 0), (0, npad)))[None]

    ncores = 2 if b % 2 == 0 else 1
    bb = b // ncores
    f32 = jnp.float32
    outs = pl.pallas_call(
        _nms_kernel,
        grid=(ncores,),
        in_specs=[
            pl.BlockSpec((bb, c, nn), lambda i: (i, 0, 0)),
            pl.BlockSpec((bb, 4, nn), lambda i: (i, 0, 0)),
            pl.BlockSpec((1, 4, nn), lambda i: (0, 0, 0)),
        ],
        out_specs=[pl.BlockSpec((1, bb, _K), lambda i: (i, 0, 0))] * 6,
        out_shape=[jax.ShapeDtypeStruct((ncores, bb, _K), f32)] * 6,
        scratch_shapes=[
            pltpu.VMEM((bb, c, nn), f32),
            pltpu.VMEM((bb, c, _K), f32),
            pltpu.VMEM((bb, c, _K), f32),
            pltpu.VMEM((bb, c, _K), f32),
            pltpu.VMEM((bb, c, _K), f32),
            pltpu.VMEM((bb, c, _K), f32),
        ],
        compiler_params=pltpu.CompilerParams(dimension_semantics=("parallel",)),
    )(labels_t, deltas_t, priors_t)
    y1, x1, y2, x2, scores, cls = [o.reshape(b, _K) for o in outs]
    boxes = jnp.stack([y1, x1, y2, x2], axis=-1)
    return boxes, scores, cls


# speedup vs baseline: 30.8052x; 1.1248x over previous
import jax
import jax.numpy as jnp
from jax.experimental import pallas as pl
from jax.experimental.pallas import tpu as pltpu

_K = 200
_MW = 32
_IOU_THR = 0.5
_SCORE_THR = 0.5
_VAR = (0.1, 0.1, 0.2, 0.2)
_NEG = float("-inf")


def _nms_kernel(labels_ref, deltas_ref, priors_ref,
                y1o, x1o, y2o, x2o, sco, clso,
                sc_ref, ky_ref, sy1, sx1, sy2, sx2):
    bb, c, n = labels_ref.shape

    py1 = priors_ref[:, 0:1, :]
    px1 = priors_ref[:, 1:2, :]
    py2 = priors_ref[:, 2:3, :]
    px2 = priors_ref[:, 3:4, :]
    anc_h = py2 - py1
    anc_w = px2 - px1
    ctr_y = py1 + 0.5 * anc_h
    ctr_x = px1 + 0.5 * anc_w
    d0 = deltas_ref[:, 0:1, :] * _VAR[0]
    d1 = deltas_ref[:, 1:2, :] * _VAR[1]
    d2 = deltas_ref[:, 2:3, :] * _VAR[2]
    d3 = deltas_ref[:, 3:4, :] * _VAR[3]
    bh = jnp.exp(d2) * anc_h
    bw = jnp.exp(d3) * anc_w
    bcy = d0 * anc_h + ctr_y
    bcx = d1 * anc_w + ctr_x
    y1 = bcy - 0.5 * bh
    x1 = bcx - 0.5 * bw
    y2 = y1 + bh
    x2 = x1 + bw
    areas = (y2 - y1) * (x2 - x1)

    labels = labels_ref[...]
    colmax = jnp.max(labels, axis=1, keepdims=True)
    bg = labels[:, 0:1, :] >= colmax
    s = jnp.where(bg, 0.0, labels)
    sc_ref[...] = jnp.where(s > _SCORE_THR, s, _NEG)

    lane = jax.lax.broadcasted_iota(jnp.int32, (bb, c, n), 2)
    slot = jax.lax.broadcasted_iota(jnp.int32, (bb, c, _K), 2)

    ky_ref[...] = jnp.full((bb, c, _K), -1.0, jnp.float32)
    sy1[...] = jnp.zeros((bb, c, _K), jnp.float32)
    sx1[...] = jnp.zeros((bb, c, _K), jnp.float32)
    sy2[...] = jnp.zeros((bb, c, _K), jnp.float32)
    sx2[...] = jnp.zeros((bb, c, _K), jnp.float32)

    def nms_cond(carry):
        t, done = carry
        return jnp.logical_and(t < _K, jnp.logical_not(done))

    def one_sel(sc, t):
        m = jnp.max(sc, axis=2, keepdims=True)
        valid = m > _NEG
        idx = jnp.min(jnp.where(sc == m, lane, n), axis=2, keepdims=True)
        onehot = lane == idx
        by1 = jnp.sum(jnp.where(onehot, y1, 0.0), axis=2, keepdims=True)
        bx1 = jnp.sum(jnp.where(onehot, x1, 0.0), axis=2, keepdims=True)
        by2 = jnp.sum(jnp.where(onehot, y2, 0.0), axis=2, keepdims=True)
        bx2 = jnp.sum(jnp.where(onehot, x2, 0.0), axis=2, keepdims=True)
        a = (by2 - by1) * (bx2 - bx1)
        yy1 = jnp.maximum(by1, y1)
        xx1 = jnp.maximum(bx1, x1)
        yy2 = jnp.minimum(by2, y2)
        xx2 = jnp.minimum(bx2, x2)
        inter = jnp.maximum(yy2 - yy1, 0.0) * jnp.maximum(xx2 - xx1, 0.0)
        iou = inter / (a + areas - inter + 1e-8)
        sc = jnp.where((iou > _IOU_THR) | onehot, _NEG, sc)
        wmask = (slot == t) & valid
        ky_ref[...] = jnp.where(wmask, m, ky_ref[...])
        sy1[...] = jnp.where(wmask, by1, sy1[...])
        sx1[...] = jnp.where(wmask, bx1, sx1[...])
        sy2[...] = jnp.where(wmask, by2, sy2[...])
        sx2[...] = jnp.where(wmask, bx2, sx2[...])
        return sc, m

    def nms_body(carry):
        t, _ = carry
        sc = sc_ref[...]
        sc, _ = one_sel(sc, t)
        sc, m2 = one_sel(sc, t + 1)
        sc_ref[...] = sc
        big_m = jnp.max(m2, axis=1, keepdims=True)
        cnt = jnp.sum((ky_ref[...] > big_m).astype(jnp.int32),
                      axis=(1, 2), keepdims=True)
        batch_done = (cnt >= _K) | (big_m <= _NEG)
        return t + 2, jnp.all(batch_done)

    t_stop, _ = jax.lax.while_loop(nms_cond, nms_body, (0, False))

    w = min(_MW, _K)
    sl = 1
    while sl < c:
        sl *= 2
    nfl = sl * w
    s_io = jax.lax.broadcasted_iota(jnp.int32, (bb, sl, w), 1)
    l_io = jax.lax.broadcasted_iota(jnp.int32, (bb, sl, w), 2)
    ffl = s_io * w + l_io
    fi3 = s_io * _K + l_io

    def padrows(x, fill):
        if sl == c:
            return x
        return jnp.concatenate(
            [x, jnp.full((bb, sl - c, w), fill, jnp.float32)], axis=1)

    key = padrows(ky_ref[:, :, :w], -1.0)
    pv1 = padrows(sy1[:, :, :w], 0.0)
    px1 = padrows(sx1[:, :, :w], 0.0)
    pv2 = padrows(sy2[:, :, :w], 0.0)
    px2 = padrows(sx2[:, :, :w], 0.0)
    fis = fi3

    k = 2
    while k <= nfl:
        j = k // 2
        while j >= 1:
            if j < w:
                ax, sh, lower = 2, j, (l_io & j) == 0
            else:
                ax, sh, lower = 1, j // w, (s_io & (j // w)) == 0
            seg0 = (ffl & k) == 0

            def ex(x, _ax=ax, _sh=sh, _lo=lower):
                return jnp.where(_lo, jnp.roll(x, -_sh, axis=_ax),
                                 jnp.roll(x, _sh, axis=_ax))

            okey, ofis = ex(key), ex(fis)
            g = (key > okey) | ((key == okey) & (fis < ofis))
            sel_mine = (seg0 == lower) == g
            key = jnp.where(sel_mine, key, okey)
            fis = jnp.where(sel_mine, fis, ofis)
            pv1 = jnp.where(sel_mine, pv1, ex(pv1))
            px1 = jnp.where(sel_mine, px1, ex(px1))
            pv2 = jnp.where(sel_mine, pv2, ex(pv2))
            px2 = jnp.where(sel_mine, px2, ex(px2))
            j //= 2
        k *= 2

    fast = t_stop <= w
    fvalid = key > 0.0
    fsc = jnp.where(fvalid, key, 0.0)
    fcls = jnp.where(fvalid, (fis // _K).astype(jnp.float32), 0.0)
    row = 0
    off = 0
    while off < _K:
        ln = min(w, _K - off)
        osl = slice(off, off + ln)
        sco[0, :, osl] = jnp.where(fast, fsc[:, row, :ln], sco[0, :, osl])
        clso[0, :, osl] = jnp.where(fast, fcls[:, row, :ln], clso[0, :, osl])
        y1o[0, :, osl] = jnp.where(fast, pv1[:, row, :ln], y1o[0, :, osl])
        x1o[0, :, osl] = jnp.where(fast, px1[:, row, :ln], x1o[0, :, osl])
        y2o[0, :, osl] = jnp.where(fast, pv2[:, row, :ln], y2o[0, :, osl])
        x2o[0, :, osl] = jnp.where(fast, px2[:, row, :ln], x2o[0, :, osl])
        row += 1
        off += ln

    fi = (jax.lax.broadcasted_iota(jnp.int32, (bb, c, _K), 1) * _K
          + jax.lax.broadcasted_iota(jnp.int32, (bb, c, _K), 2))
    oslot = jax.lax.broadcasted_iota(jnp.int32, (1, bb, _K), 2)

    unroll = 4
    tv1, tx1, tv2, tx2 = sy1[...], sx1[...], sy2[...], sx2[...]

    def merge_body(i, carry):
        ky = ky_ref[...]
        for u in range(unroll):
            t = i * unroll + u
            m = jnp.max(ky, axis=(1, 2), keepdims=True)
            pos = jnp.min(jnp.where(ky == m, fi, c * _K), axis=(1, 2), keepdims=True)
            oh = fi == pos
            valid = m > 0.0
            omask = oslot == t
            sco[...] = jnp.where(omask, jnp.where(valid, m, 0.0)[:, :, 0][None], sco[...])
            cls = (pos // _K).astype(jnp.float32)
            clso[...] = jnp.where(omask, jnp.where(valid, cls, 0.0)[:, :, 0][None], clso[...])
            by1 = jnp.sum(jnp.where(oh, tv1, 0.0), axis=(1, 2), keepdims=True)
            bx1 = jnp.sum(jnp.where(oh, tx1, 0.0), axis=(1, 2), keepdims=True)
            by2 = jnp.sum(jnp.where(oh, tv2, 0.0), axis=(1, 2), keepdims=True)
            bx2 = jnp.sum(jnp.where(oh, tx2, 0.0), axis=(1, 2), keepdims=True)
            y1o[...] = jnp.where(omask, by1[:, :, 0][None], y1o[...])
            x1o[...] = jnp.where(omask, bx1[:, :, 0][None], x1o[...])
            y2o[...] = jnp.where(omask, by2[:, :, 0][None], y2o[...])
            x2o[...] = jnp.where(omask, bx2[:, :, 0][None], x2o[...])
            ky = jnp.where(oh, -2.0, ky)
        ky_ref[...] = ky
        return carry

    jax.lax.fori_loop(0, jnp.where(fast, 0, _K // unroll), merge_body, 0)


def kernel(pred_deltas, pred_labels, prior_boxes):
    b, n, c = pred_labels.shape
    npad = -n % 128
    nn = n + npad
    labels_t = jnp.pad(pred_labels.transpose(0, 2, 1), ((0, 0), (0, 0), (0, npad)))
    deltas_t = jnp.pad(pred_deltas.transpose(0, 2, 1), ((0, 0), (0, 0), (0, npad)))
    priors_t = jnp.pad(prior_boxes.T, ((0, 0), (0, npad)))[None]

    ncores = 2 if b % 2 == 0 else 1
    bb = b // ncores
    f32 = jnp.float32
    outs = pl.pallas_call(
        _nms_kernel,
        grid=(ncores,),
        in_specs=[
            pl.BlockSpec((bb, c, nn), lambda i: (i, 0, 0)),
            pl.BlockSpec((bb, 4, nn), lambda i: (i, 0, 0)),
            pl.BlockSpec((1, 4, nn), lambda i: (0, 0, 0)),
        ],
        out_specs=[pl.BlockSpec((1, bb, _K), lambda i: (i, 0, 0))] * 6,
        out_shape=[jax.ShapeDtypeStruct((ncores, bb, _K), f32)] * 6,
        scratch_shapes=[
            pltpu.VMEM((bb, c, nn), f32),
            pltpu.VMEM((bb, c, _K), f32),
            pltpu.VMEM((bb, c, _K), f32),
            pltpu.VMEM((bb, c, _K), f32),
            pltpu.VMEM((bb, c, _K), f32),
            pltpu.VMEM((bb, c, _K), f32),
        ],
        compiler_params=pltpu.CompilerParams(dimension_semantics=("parallel",)),
    )(labels_t, deltas_t, priors_t)
    y1, x1, y2, x2, scores, cls = [o.reshape(b, _K) for o in outs]
    boxes = jnp.stack([y1, x1, y2, x2], axis=-1)
    return boxes, scores, cls


# speedup vs baseline: 32.0284x; 1.0397x over previous
import jax
import jax.numpy as jnp
from jax.experimental import pallas as pl
from jax.experimental.pallas import tpu as pltpu

_K = 200
_MW = 32
_IOU_THR = 0.5
_SCORE_THR = 0.5
_VAR = (0.1, 0.1, 0.2, 0.2)
_NEG = float("-inf")


def _nms_kernel(labels_ref, deltas_ref, priors_ref,
                y1o, x1o, y2o, x2o, sco, clso,
                sc_ref, ky_ref, sy1, sx1, sy2, sx2):
    bb, c, n = labels_ref.shape

    py1 = priors_ref[:, 0:1, :]
    px1 = priors_ref[:, 1:2, :]
    py2 = priors_ref[:, 2:3, :]
    px2 = priors_ref[:, 3:4, :]
    anc_h = py2 - py1
    anc_w = px2 - px1
    ctr_y = py1 + 0.5 * anc_h
    ctr_x = px1 + 0.5 * anc_w
    d0 = deltas_ref[:, 0:1, :] * _VAR[0]
    d1 = deltas_ref[:, 1:2, :] * _VAR[1]
    d2 = deltas_ref[:, 2:3, :] * _VAR[2]
    d3 = deltas_ref[:, 3:4, :] * _VAR[3]
    bh = jnp.exp(d2) * anc_h
    bw = jnp.exp(d3) * anc_w
    bcy = d0 * anc_h + ctr_y
    bcx = d1 * anc_w + ctr_x
    y1 = bcy - 0.5 * bh
    x1 = bcx - 0.5 * bw
    y2 = y1 + bh
    x2 = x1 + bw
    areas = (y2 - y1) * (x2 - x1)

    labels = labels_ref[...]
    colmax = jnp.max(labels, axis=1, keepdims=True)
    bg = labels[:, 0:1, :] >= colmax
    s = jnp.where(bg, 0.0, labels)
    sc_ref[...] = jnp.where(s > _SCORE_THR, s, _NEG)

    lane = jax.lax.broadcasted_iota(jnp.int32, (bb, c, n), 2)
    slot = jax.lax.broadcasted_iota(jnp.int32, (bb, c, _K), 2)

    ky_ref[...] = jnp.full((bb, c, _K), -1.0, jnp.float32)
    sy1[...] = jnp.zeros((bb, c, _K), jnp.float32)
    sx1[...] = jnp.zeros((bb, c, _K), jnp.float32)
    sy2[...] = jnp.zeros((bb, c, _K), jnp.float32)
    sx2[...] = jnp.zeros((bb, c, _K), jnp.float32)

    def nms_cond(carry):
        t, done = carry
        return jnp.logical_and(t < _K, jnp.logical_not(done))

    def one_sel(sc, t):
        m = jnp.max(sc, axis=2, keepdims=True)
        valid = m > _NEG
        idx = jnp.min(jnp.where(sc == m, lane, n), axis=2, keepdims=True)
        onehot = lane == idx
        by1 = jnp.sum(jnp.where(onehot, y1, 0.0), axis=2, keepdims=True)
        bx1 = jnp.sum(jnp.where(onehot, x1, 0.0), axis=2, keepdims=True)
        by2 = jnp.sum(jnp.where(onehot, y2, 0.0), axis=2, keepdims=True)
        bx2 = jnp.sum(jnp.where(onehot, x2, 0.0), axis=2, keepdims=True)
        a = (by2 - by1) * (bx2 - bx1)
        yy1 = jnp.maximum(by1, y1)
        xx1 = jnp.maximum(bx1, x1)
        yy2 = jnp.minimum(by2, y2)
        xx2 = jnp.minimum(bx2, x2)
        inter = jnp.maximum(yy2 - yy1, 0.0) * jnp.maximum(xx2 - xx1, 0.0)
        iou = inter / (a + areas - inter + 1e-8)
        sc = jnp.where((iou > _IOU_THR) | onehot, _NEG, sc)
        wmask = (slot == t) & valid
        ky_ref[...] = jnp.where(wmask, m, ky_ref[...])
        sy1[...] = jnp.where(wmask, by1, sy1[...])
        sx1[...] = jnp.where(wmask, bx1, sx1[...])
        sy2[...] = jnp.where(wmask, by2, sy2[...])
        sx2[...] = jnp.where(wmask, bx2, sx2[...])
        return sc, m

    def nms_body(carry):
        t, _ = carry
        sc = sc_ref[...]
        sc, _ = one_sel(sc, t)
        sc, m2 = one_sel(sc, t + 1)
        sc_ref[...] = sc
        big_m = jnp.max(m2, axis=1, keepdims=True)
        cnt = jnp.sum((ky_ref[...] > big_m).astype(jnp.int32),
                      axis=(1, 2), keepdims=True)
        batch_done = (cnt >= _K) | (big_m <= _NEG)
        return t + 2, jnp.all(batch_done)

    t_stop, _ = jax.lax.while_loop(nms_cond, nms_body, (0, False))

    w = min(_MW, _K)
    sl = 1
    while sl < c:
        sl *= 2
    nfl = sl * w
    s_io = jax.lax.broadcasted_iota(jnp.int32, (bb, sl, w), 1)
    l_io = jax.lax.broadcasted_iota(jnp.int32, (bb, sl, w), 2)
    ffl = s_io * w + l_io
    fi3 = s_io * _K + l_io

    def padrows(x, fill):
        if sl == c:
            return x
        return jnp.concatenate(
            [x, jnp.full((bb, sl - c, w), fill, jnp.float32)], axis=1)

    key = padrows(ky_ref[:, :, :w], -1.0)
    pv1 = padrows(sy1[:, :, :w], 0.0)
    px1 = padrows(sx1[:, :, :w], 0.0)
    pv2 = padrows(sy2[:, :, :w], 0.0)
    px2 = padrows(sx2[:, :, :w], 0.0)
    fis = fi3

    odd = (s_io & 1) == 1
    jj = 1
    while jj < w:
        lo = (l_io & jj) == 0

        def exr(x, _j=jj, _lo=lo):
            return jnp.where(_lo, jnp.roll(x, -_j, axis=2),
                             jnp.roll(x, _j, axis=2))

        key = jnp.where(odd, exr(key), key)
        fis = jnp.where(odd, exr(fis), fis)
        pv1 = jnp.where(odd, exr(pv1), pv1)
        px1 = jnp.where(odd, exr(px1), px1)
        pv2 = jnp.where(odd, exr(pv2), pv2)
        px2 = jnp.where(odd, exr(px2), px2)
        jj *= 2

    k = 2 * w
    while k <= nfl:
        j = k // 2
        while j >= 1:
            if j < w:
                ax, sh, lower = 2, j, (l_io & j) == 0
            else:
                ax, sh, lower = 1, j // w, (s_io & (j // w)) == 0
            seg0 = (ffl & k) == 0

            def ex(x, _ax=ax, _sh=sh, _lo=lower):
                return jnp.where(_lo, jnp.roll(x, -_sh, axis=_ax),
                                 jnp.roll(x, _sh, axis=_ax))

            okey, ofis = ex(key), ex(fis)
            g = (key > okey) | ((key == okey) & (fis < ofis))
            sel_mine = (seg0 == lower) == g
            key = jnp.where(sel_mine, key, okey)
            fis = jnp.where(sel_mine, fis, ofis)
            pv1 = jnp.where(sel_mine, pv1, ex(pv1))
            px1 = jnp.where(sel_mine, px1, ex(px1))
            pv2 = jnp.where(sel_mine, pv2, ex(pv2))
            px2 = jnp.where(sel_mine, px2, ex(px2))
            j //= 2
        k *= 2

    fast = t_stop <= w
    fvalid = key > 0.0
    fsc = jnp.where(fvalid, key, 0.0)
    fcls = jnp.where(fvalid, (fis // _K).astype(jnp.float32), 0.0)
    row = 0
    off = 0
    while off < _K:
        ln = min(w, _K - off)
        osl = slice(off, off + ln)
        sco[0, :, osl] = jnp.where(fast, fsc[:, row, :ln], sco[0, :, osl])
        clso[0, :, osl] = jnp.where(fast, fcls[:, row, :ln], clso[0, :, osl])
        y1o[0, :, osl] = jnp.where(fast, pv1[:, row, :ln], y1o[0, :, osl])
        x1o[0, :, osl] = jnp.where(fast, px1[:, row, :ln], x1o[0, :, osl])
        y2o[0, :, osl] = jnp.where(fast, pv2[:, row, :ln], y2o[0, :, osl])
        x2o[0, :, osl] = jnp.where(fast, px2[:, row, :ln], x2o[0, :, osl])
        row += 1
        off += ln

    fi = (jax.lax.broadcasted_iota(jnp.int32, (bb, c, _K), 1) * _K
          + jax.lax.broadcasted_iota(jnp.int32, (bb, c, _K), 2))
    oslot = jax.lax.broadcasted_iota(jnp.int32, (1, bb, _K), 2)

    unroll = 4
    tv1, tx1, tv2, tx2 = sy1[...], sx1[...], sy2[...], sx2[...]

    def merge_body(i, carry):
        ky = ky_ref[...]
        for u in range(unroll):
            t = i * unroll + u
            m = jnp.max(ky, axis=(1, 2), keepdims=True)
            pos = jnp.min(jnp.where(ky == m, fi, c * _K), axis=(1, 2), keepdims=True)
            oh = fi == pos
            valid = m > 0.0
            omask = oslot == t
            sco[...] = jnp.where(omask, jnp.where(valid, m, 0.0)[:, :, 0][None], sco[...])
            cls = (pos // _K).astype(jnp.float32)
            clso[...] = jnp.where(omask, jnp.where(valid, cls, 0.0)[:, :, 0][None], clso[...])
            by1 = jnp.sum(jnp.where(oh, tv1, 0.0), axis=(1, 2), keepdims=True)
            bx1 = jnp.sum(jnp.where(oh, tx1, 0.0), axis=(1, 2), keepdims=True)
            by2 = jnp.sum(jnp.where(oh, tv2, 0.0), axis=(1, 2), keepdims=True)
            bx2 = jnp.sum(jnp.where(oh, tx2, 0.0), axis=(1, 2), keepdims=True)
            y1o[...] = jnp.where(omask, by1[:, :, 0][None], y1o[...])
            x1o[...] = jnp.where(omask, bx1[:, :, 0][None], x1o[...])
            y2o[...] = jnp.where(omask, by2[:, :, 0][None], y2o[...])
            x2o[...] = jnp.where(omask, bx2[:, :, 0][None], x2o[...])
            ky = jnp.where(oh, -2.0, ky)
        ky_ref[...] = ky
        return carry

    jax.lax.fori_loop(0, jnp.where(fast, 0, _K // unroll), merge_body, 0)


def kernel(pred_deltas, pred_labels, prior_boxes):
    b, n, c = pred_labels.shape
    npad = -n % 128
    nn = n + npad
    labels_t = jnp.pad(pred_labels.transpose(0, 2, 1), ((0, 0), (0, 0), (0, npad)))
    deltas_t = jnp.pad(pred_deltas.transpose(0, 2, 1), ((0, 0), (0, 0), (0, npad)))
    priors_t = jnp.pad(prior_boxes.T, ((0, 0), (0, npad)))[None]

    ncores = 2 if b % 2 == 0 else 1
    bb = b // ncores
    f32 = jnp.float32
    outs = pl.pallas_call(
        _nms_kernel,
        grid=(ncores,),
        in_specs=[
            pl.BlockSpec((bb, c, nn), lambda i: (i, 0, 0)),
            pl.BlockSpec((bb, 4, nn), lambda i: (i, 0, 0)),
            pl.BlockSpec((1, 4, nn), lambda i: (0, 0, 0)),
        ],
        out_specs=[pl.BlockSpec((1, bb, _K), lambda i: (i, 0, 0))] * 6,
        out_shape=[jax.ShapeDtypeStruct((ncores, bb, _K), f32)] * 6,
        scratch_shapes=[
            pltpu.VMEM((bb, c, nn), f32),
            pltpu.VMEM((bb, c, _K), f32),
            pltpu.VMEM((bb, c, _K), f32),
            pltpu.VMEM((bb, c, _K), f32),
            pltpu.VMEM((bb, c, _K), f32),
            pltpu.VMEM((bb, c, _K), f32),
        ],
        compiler_params=pltpu.CompilerParams(dimension_semantics=("parallel",)),
    )(labels_t, deltas_t, priors_t)
    y1, x1, y2, x2, scores, cls = [o.reshape(b, _K) for o in outs]
    boxes = jnp.stack([y1, x1, y2, x2], axis=-1)
    return boxes, scores, cls
